# baseline (speedup 1.0000x reference)
import functools

import jax
import jax.numpy as jnp
from jax import lax
from jax.experimental import pallas as pl
from jax.experimental.pallas import tpu as pltpu
from jax.experimental.pallas import tpu_sc as plsc

_NC, _NS, _LANES = 2, 16, 16
_NW = _NC * _NS


def _make_gather(n_idx: int, table_size: int):
  per_w = n_idx // _NW
  assert per_w * _NW == n_idx and per_w % 8 == 0
  chunk = 8000
  if per_w % chunk != 0:
    chunk = per_w
  n_chunks = per_w // chunk
  assert chunk % _LANES == 0

  mesh = plsc.VectorSubcoreMesh(
      core_axis_name="c", subcore_axis_name="s",
      num_cores=_NC, num_subcores=_NS)

  @functools.partial(
      pl.kernel,
      out_type=jax.ShapeDtypeStruct((n_idx,), jnp.float32),
      mesh=mesh,
      scratch_types=[
          pltpu.VMEM((table_size,), jnp.float32),
          pltpu.VMEM((chunk,), jnp.int32),
          pltpu.VMEM((chunk,), jnp.float32),
      ],
      compiler_params=pltpu.CompilerParams(needs_layout_passes=False),
  )
  def gather_kernel(table_hbm, idx_hbm, out_hbm, table_v, idx_v, out_v):
    wid = lax.axis_index("s") * _NC + lax.axis_index("c")
    pltpu.sync_copy(table_hbm, table_v)
    base = wid * per_w
    for c in range(n_chunks):
      off = base + c * chunk
      pltpu.sync_copy(idx_hbm.at[pl.ds(off, chunk)], idx_v)

      @plsc.parallel_loop(0, chunk, _LANES, unroll=8)
      def _(i):
        ids = idx_v[pl.ds(i, _LANES)]
        out_v[pl.ds(i, _LANES)] = plsc.load_gather(table_v, [ids])

      pltpu.sync_copy(out_v, out_hbm.at[pl.ds(off, chunk)])

  return gather_kernel


def _render_body(ray_ref, zbuf_ref, sigg_ref, w1_ref, b1_ref, w2big_ref,
                 b2t_ref, s24_ref, r24_ref,
                 color_ref, acc_ref, depth_ref, dmax_ref, *, k_samples):
  kk = k_samples
  hidden = w1_ref.shape[1]
  ray = ray_ref[...]
  odirs = ray[:, :6]
  dirs = ray[:, 3:6]
  cos = ray[:, 6:7]
  z = zbuf_ref[...]
  t = z / cos

  w1 = w1_ref[...]
  ad = jnp.dot(odirs, w1, preferred_element_type=jnp.float32) + b1_ref[...]
  d3 = jnp.dot(dirs, w1[:3], preferred_element_type=jnp.float32)

  bs = ray.shape[0]
  adb = ad.astype(jnp.bfloat16)
  d3b = d3.astype(jnp.bfloat16)
  tb = t.astype(jnp.bfloat16)
  pre = jnp.zeros((bs, 3 * kk), jnp.float32)
  for k in range(kk):
    h = jnp.maximum(adb + d3b * tb[:, k:k + 1], 0.0)
    pre = pre + jnp.dot(h, w2big_ref[k * hidden:(k + 1) * hidden, :],
                        preferred_element_type=jnp.float32)
  cfull = jax.nn.sigmoid(pre + b2t_ref[...])

  s = jax.nn.sigmoid(sigg_ref[...])
  s = jnp.where(z > 0, s, 0.0)

  f = 1.0 - s + 1e-10
  one = jnp.ones((bs, 1), jnp.float32)
  x = jnp.concatenate([one, f[:, :kk - 1]], axis=1)
  d = 1
  while d < kk:
    x = x * jnp.concatenate(
        [jnp.ones((bs, d), jnp.float32), x[:, :kk - d]], axis=1)
    d *= 2
  w8 = s * x

  acc = jnp.sum(w8, axis=1, keepdims=True)
  depth = jnp.sum(w8 * z, axis=1, keepdims=True)
  wexp = jnp.dot(w8, r24_ref[...], preferred_element_type=jnp.float32)
  wc = wexp * cfull
  color = jnp.dot(wc, s24_ref[...], preferred_element_type=jnp.float32)
  color_ref[...] = color + (1.0 - acc)
  acc_ref[...] = acc
  depth_ref[...] = depth

  blk_max = jnp.max(depth)
  @pl.when(pl.program_id(0) == 0)
  def _():
    dmax_ref[...] = jnp.full(dmax_ref.shape, -jnp.inf, jnp.float32)
  dmax_ref[...] = jnp.maximum(dmax_ref[...], blk_max)


def _norm_body(d_ref, dmax_ref, out_ref):
  m = jnp.max(dmax_ref[...])
  out_ref[...] = (d_ref[...] - 2.0) / (m - 2.0)


def kernel(zbuf, ray, idx, sigma, W1, b1, W2, b2):
  B, H, W, K = idx.shape
  n = B * H * W
  zb = zbuf.reshape(n, K)
  rayf = ray.reshape(n, 7)
  idxf = idx.reshape(n * K)
  table = sigma.reshape(-1)

  sigg = _make_gather(n * K, table.shape[0])(table, idxf).reshape(n, K)

  hidden = W1.shape[1]
  eye_k = jnp.eye(K, dtype=jnp.float32)
  w2big = jnp.kron(eye_k, W2).astype(jnp.bfloat16)
  b2t = jnp.tile(b2.reshape(1, 3), (1, K))
  s24 = jnp.tile(jnp.eye(3, dtype=jnp.float32), (K, 1))
  r24 = jnp.repeat(eye_k, 3, axis=1)

  bs = 4000
  assert n % bs == 0
  grid = (n // bs,)
  row_spec = lambda d: pl.BlockSpec((bs, d), lambda i: (i, 0))
  full_spec = lambda a, b: pl.BlockSpec((a, b), lambda i: (0, 0))
  color = pl.pallas_call(
      functools.partial(_render_body, k_samples=K),
      grid=grid,
      in_specs=[
          row_spec(7), row_spec(K), row_spec(K),
          full_spec(6, hidden), full_spec(1, hidden),
          full_spec(K * hidden, 3 * K), full_spec(1, 3 * K),
          full_spec(3 * K, 3), full_spec(K, 3 * K),
      ],
      out_specs=[row_spec(3), row_spec(1), row_spec(1),
                 pl.BlockSpec((8, 128), lambda i: (0, 0))],
      out_shape=[
          jax.ShapeDtypeStruct((n, 3), jnp.float32),
          jax.ShapeDtypeStruct((n, 1), jnp.float32),
          jax.ShapeDtypeStruct((n, 1), jnp.float32),
          jax.ShapeDtypeStruct((8, 128), jnp.float32),
      ],
  )(rayf, zb, sigg, W1, b1.reshape(1, hidden), w2big, b2t, s24, r24)
  color, acc, depth_raw, dmax = color

  depth = pl.pallas_call(
      _norm_body,
      grid=grid,
      in_specs=[row_spec(1), pl.BlockSpec((8, 128), lambda i: (0, 0))],
      out_specs=row_spec(1),
      out_shape=jax.ShapeDtypeStruct((n, 1), jnp.float32),
  )(depth_raw, dmax)

  return (color.reshape(B, H, W, 3), acc.reshape(B, H, W, 1),
          depth.reshape(B, H, W, 1))


# speedup vs baseline: 1.0869x; 1.0869x over previous
import functools

import jax
import jax.numpy as jnp
from jax import lax
from jax.experimental import pallas as pl
from jax.experimental.pallas import tpu as pltpu
from jax.experimental.pallas import tpu_sc as plsc

_NC, _NS, _LANES = 2, 16, 16
_NW = _NC * _NS


def _make_gather(n_idx: int, table_size: int):
  per_w = n_idx // _NW
  assert per_w * _NW == n_idx and per_w % 8 == 0
  chunk = 8000
  if per_w % chunk != 0:
    chunk = per_w
  n_chunks = per_w // chunk
  assert chunk % _LANES == 0

  mesh = plsc.VectorSubcoreMesh(
      core_axis_name="c", subcore_axis_name="s",
      num_cores=_NC, num_subcores=_NS)

  @functools.partial(
      pl.kernel,
      out_type=jax.ShapeDtypeStruct((n_idx,), jnp.float32),
      mesh=mesh,
      scratch_types=[
          pltpu.VMEM((table_size,), jnp.float32),
          pltpu.VMEM((chunk,), jnp.int32),
          pltpu.VMEM((chunk,), jnp.float32),
      ],
      compiler_params=pltpu.CompilerParams(needs_layout_passes=False),
  )
  def gather_kernel(table_hbm, idx_hbm, out_hbm, table_v, idx_v, out_v):
    wid = lax.axis_index("s") * _NC + lax.axis_index("c")
    pltpu.sync_copy(table_hbm, table_v)
    base = wid * per_w
    for c in range(n_chunks):
      off = base + c * chunk
      pltpu.sync_copy(idx_hbm.at[pl.ds(off, chunk)], idx_v)

      @plsc.parallel_loop(0, chunk, _LANES, unroll=8)
      def _(i):
        ids = idx_v[pl.ds(i, _LANES)]
        out_v[pl.ds(i, _LANES)] = plsc.load_gather(table_v, [ids])

      pltpu.sync_copy(out_v, out_hbm.at[pl.ds(off, chunk)])

  return gather_kernel


def _render_body(ray_ref, zbuf_ref, sigg_ref, w1_ref, b1_ref, w2big_ref,
                 b2t_ref, s24_ref, r24_ref,
                 color_ref, acc_ref, depth_ref, *, k_samples):
  kk = k_samples
  hidden = w1_ref.shape[1]
  ray = ray_ref[...]
  odirs = ray[:, :6]
  dirs = ray[:, 3:6]
  cos = ray[:, 6:7]
  z = zbuf_ref[...]
  t = z / cos

  w1 = w1_ref[...]
  ad = jnp.dot(odirs, w1, preferred_element_type=jnp.float32) + b1_ref[...]
  d3 = jnp.dot(dirs, w1[:3], preferred_element_type=jnp.float32)

  bs = ray.shape[0]
  adb = ad.astype(jnp.bfloat16)
  d3b = d3.astype(jnp.bfloat16)
  tb = t.astype(jnp.bfloat16)
  pre = jnp.zeros((bs, 3 * kk), jnp.float32)
  for k in range(kk):
    h = jnp.maximum(adb + d3b * tb[:, k:k + 1], 0.0)
    pre = pre + jnp.dot(h, w2big_ref[k * hidden:(k + 1) * hidden, :],
                        preferred_element_type=jnp.float32)
  cfull = jax.nn.sigmoid(pre + b2t_ref[...])

  s = jax.nn.sigmoid(sigg_ref[...])
  s = jnp.where(z > 0, s, 0.0)

  f = 1.0 - s + 1e-10
  one = jnp.ones((bs, 1), jnp.float32)
  x = jnp.concatenate([one, f[:, :kk - 1]], axis=1)
  d = 1
  while d < kk:
    x = x * jnp.concatenate(
        [jnp.ones((bs, d), jnp.float32), x[:, :kk - d]], axis=1)
    d *= 2
  w8 = s * x

  acc = jnp.sum(w8, axis=1, keepdims=True)
  depth = jnp.sum(w8 * z, axis=1, keepdims=True)
  wexp = jnp.dot(w8, r24_ref[...], preferred_element_type=jnp.float32)
  wc = wexp * cfull
  color = jnp.dot(wc, s24_ref[...], preferred_element_type=jnp.float32)
  color_ref[...] = color + (1.0 - acc)
  acc_ref[...] = acc
  depth_ref[...] = depth


def _norm_body(d_ref, out_ref):
  d = d_ref[...]
  out_ref[...] = (d - 2.0) / (jnp.max(d) - 2.0)


def kernel(zbuf, ray, idx, sigma, W1, b1, W2, b2):
  B, H, W, K = idx.shape
  n = B * H * W
  zb = zbuf.reshape(n, K)
  rayf = ray.reshape(n, 7)
  idxf = idx.reshape(n * K)
  table = sigma.reshape(-1)

  sigg = _make_gather(n * K, table.shape[0])(table, idxf).reshape(n, K)

  hidden = W1.shape[1]
  eye_k = jnp.eye(K, dtype=jnp.float32)
  w2big = jnp.kron(eye_k, W2).astype(jnp.bfloat16)
  b2t = jnp.tile(b2.reshape(1, 3), (1, K))
  s24 = jnp.tile(jnp.eye(3, dtype=jnp.float32), (K, 1))
  r24 = jnp.repeat(eye_k, 3, axis=1)

  bs = 4000
  assert n % bs == 0
  grid = (n // bs,)
  row_spec = lambda d: pl.BlockSpec((bs, d), lambda i: (i, 0))
  full_spec = lambda a, b: pl.BlockSpec((a, b), lambda i: (0, 0))
  color, acc, depth_raw = pl.pallas_call(
      functools.partial(_render_body, k_samples=K),
      grid=grid,
      in_specs=[
          row_spec(7), row_spec(K), row_spec(K),
          full_spec(6, hidden), full_spec(1, hidden),
          full_spec(K * hidden, 3 * K), full_spec(1, 3 * K),
          full_spec(3 * K, 3), full_spec(K, 3 * K),
      ],
      out_specs=[row_spec(3), row_spec(1), row_spec(1)],
      out_shape=[
          jax.ShapeDtypeStruct((n, 3), jnp.float32),
          jax.ShapeDtypeStruct((n, 1), jnp.float32),
          jax.ShapeDtypeStruct((n, 1), jnp.float32),
      ],
  )(rayf, zb, sigg, W1, b1.reshape(1, hidden), w2big, b2t, s24, r24)

  d2 = depth_raw.reshape(n // 128, 128)
  depth = pl.pallas_call(
      _norm_body,
      out_shape=jax.ShapeDtypeStruct(d2.shape, jnp.float32),
  )(d2)

  return (color.reshape(B, H, W, 3), acc.reshape(B, H, W, 1),
          depth.reshape(B, H, W, 1))
